# baseline (speedup 1.0000x reference)
import jax
import jax.numpy as jnp
from jax.experimental import pallas as pl
from jax.experimental.pallas import tpu as pltpu

M, K, N = 8192, 2048, 64
CHUNK = 256
NCHUNKS = M // CHUNK
NBUF = 8
NOBUF = 2


def _gate_kernel(x_hbm, w_hbm, o_hbm, xbuf, wbuf, obuf, insem, wsem, outsem):
    def in_copy(c, slot):
        return pltpu.make_async_copy(
            x_hbm.at[pl.ds(c * CHUNK, CHUNK), :],
            xbuf.at[slot],
            insem.at[slot],
        )

    def out_copy(c, oslot):
        return pltpu.make_async_copy(
            obuf.at[oslot],
            o_hbm.at[:, pl.ds(c * CHUNK, CHUNK)],
            outsem.at[oslot],
        )

    w_load = pltpu.make_async_copy(w_hbm, wbuf, wsem)
    w_load.start()
    for s in range(NBUF):
        in_copy(s, s).start()
    w_load.wait()

    w = wbuf[...].astype(jnp.bfloat16)
    for c in range(NCHUNKS):
        slot = c % NBUF
        oslot = c % NOBUF
        in_copy(c, slot).wait()
        if c >= NOBUF:
            out_copy(c - NOBUF, oslot).wait()
        obuf[oslot] = jax.lax.dot_general(
            w, xbuf[slot].astype(jnp.bfloat16), (((1,), (1,)), ((), ())),
            preferred_element_type=jnp.float32)
        out_copy(c, oslot).start()
        if c + NBUF < NCHUNKS:
            in_copy(c + NBUF, slot).start()

    for c in range(max(NCHUNKS - NOBUF, 0), NCHUNKS):
        out_copy(c, c % NOBUF).wait()


def kernel(x, W):
    out_t = pl.pallas_call(
        _gate_kernel,
        grid=(1,),
        in_specs=[
            pl.BlockSpec(memory_space=pltpu.MemorySpace.HBM),
            pl.BlockSpec(memory_space=pltpu.MemorySpace.HBM),
        ],
        out_specs=pl.BlockSpec(memory_space=pltpu.MemorySpace.HBM),
        out_shape=jax.ShapeDtypeStruct((N, M), jnp.float32),
        scratch_shapes=[
            pltpu.VMEM((NBUF, CHUNK, K), jnp.float32),
            pltpu.VMEM((N, K), jnp.float32),
            pltpu.VMEM((NOBUF, N, CHUNK), jnp.float32),
            pltpu.SemaphoreType.DMA((NBUF,)),
            pltpu.SemaphoreType.DMA,
            pltpu.SemaphoreType.DMA((NOBUF,)),
        ],
    )(x, W)
    return out_t.T


# speedup vs baseline: 1.1801x; 1.1801x over previous
import jax
import jax.numpy as jnp
from jax.experimental import pallas as pl
from jax.experimental.pallas import tpu as pltpu

M, K, N = 8192, 2048, 64
CHUNK = 512
NCHUNKS = M // CHUNK
NBUF = 5
NOBUF = 2


def _gate_kernel(x_hbm, w_hbm, o_hbm, xbuf, wbuf, obuf, insem, wsem, outsem):
    def in_copy(c, slot):
        return pltpu.make_async_copy(
            x_hbm.at[pl.ds(c * CHUNK, CHUNK), :],
            xbuf.at[slot],
            insem.at[slot],
        )

    def out_copy(c, oslot):
        return pltpu.make_async_copy(
            obuf.at[oslot],
            o_hbm.at[:, pl.ds(c * CHUNK, CHUNK)],
            outsem.at[oslot],
        )

    w_load = pltpu.make_async_copy(w_hbm, wbuf, wsem)
    w_load.start()
    for s in range(NBUF):
        in_copy(s, s).start()
    w_load.wait()

    w = wbuf[...].astype(jnp.bfloat16)
    for c in range(NCHUNKS):
        slot = c % NBUF
        oslot = c % NOBUF
        in_copy(c, slot).wait()
        if c >= NOBUF:
            out_copy(c - NOBUF, oslot).wait()
        obuf[oslot] = jax.lax.dot_general(
            w, xbuf[slot].astype(jnp.bfloat16), (((1,), (1,)), ((), ())),
            preferred_element_type=jnp.float32)
        out_copy(c, oslot).start()
        if c + NBUF < NCHUNKS:
            in_copy(c + NBUF, slot).start()

    for c in range(max(NCHUNKS - NOBUF, 0), NCHUNKS):
        out_copy(c, c % NOBUF).wait()


def kernel(x, W):
    out_t = pl.pallas_call(
        _gate_kernel,
        grid=(1,),
        in_specs=[
            pl.BlockSpec(memory_space=pltpu.MemorySpace.HBM),
            pl.BlockSpec(memory_space=pltpu.MemorySpace.HBM),
        ],
        out_specs=pl.BlockSpec(memory_space=pltpu.MemorySpace.HBM),
        out_shape=jax.ShapeDtypeStruct((N, M), jnp.float32),
        scratch_shapes=[
            pltpu.VMEM((NBUF, CHUNK, K), jnp.float32),
            pltpu.VMEM((N, K), jnp.float32),
            pltpu.VMEM((NOBUF, N, CHUNK), jnp.float32),
            pltpu.SemaphoreType.DMA((NBUF,)),
            pltpu.SemaphoreType.DMA,
            pltpu.SemaphoreType.DMA((NOBUF,)),
        ],
    )(x, W)
    return out_t.T


# speedup vs baseline: 1.1872x; 1.0060x over previous
import jax
import jax.numpy as jnp
from jax.experimental import pallas as pl
from jax.experimental.pallas import tpu as pltpu

M, K, N = 8192, 2048, 64
CHUNK = 512
NCHUNKS = M // CHUNK
NBUF = 6
NOBUF = 2


def _gate_kernel(x_hbm, w_hbm, o_hbm, xbuf, wbuf, obuf, insem, wsem, outsem):
    def in_copy(c, slot):
        return pltpu.make_async_copy(
            x_hbm.at[pl.ds(c * CHUNK, CHUNK), :],
            xbuf.at[slot],
            insem.at[slot],
        )

    def out_copy(c, oslot):
        return pltpu.make_async_copy(
            obuf.at[oslot],
            o_hbm.at[:, pl.ds(c * CHUNK, CHUNK)],
            outsem.at[oslot],
        )

    w_load = pltpu.make_async_copy(w_hbm, wbuf, wsem)
    w_load.start()
    for s in range(NBUF):
        in_copy(s, s).start()
    w_load.wait()

    w = wbuf[...].astype(jnp.bfloat16)
    for c in range(NCHUNKS):
        slot = c % NBUF
        oslot = c % NOBUF
        in_copy(c, slot).wait()
        if c >= NOBUF:
            out_copy(c - NOBUF, oslot).wait()
        obuf[oslot] = jax.lax.dot_general(
            w, xbuf[slot].astype(jnp.bfloat16), (((1,), (1,)), ((), ())),
            preferred_element_type=jnp.float32)
        out_copy(c, oslot).start()
        if c + NBUF < NCHUNKS:
            in_copy(c + NBUF, slot).start()

    for c in range(max(NCHUNKS - NOBUF, 0), NCHUNKS):
        out_copy(c, c % NOBUF).wait()


def kernel(x, W):
    out_t = pl.pallas_call(
        _gate_kernel,
        grid=(1,),
        in_specs=[
            pl.BlockSpec(memory_space=pltpu.MemorySpace.HBM),
            pl.BlockSpec(memory_space=pltpu.MemorySpace.HBM),
        ],
        out_specs=pl.BlockSpec(memory_space=pltpu.MemorySpace.HBM),
        out_shape=jax.ShapeDtypeStruct((N, M), jnp.float32),
        scratch_shapes=[
            pltpu.VMEM((NBUF, CHUNK, K), jnp.float32),
            pltpu.VMEM((N, K), jnp.float32),
            pltpu.VMEM((NOBUF, N, CHUNK), jnp.float32),
            pltpu.SemaphoreType.DMA((NBUF,)),
            pltpu.SemaphoreType.DMA,
            pltpu.SemaphoreType.DMA((NOBUF,)),
        ],
    )(x, W)
    return out_t.T


# speedup vs baseline: 1.2051x; 1.0151x over previous
import jax
import jax.numpy as jnp
from jax.experimental import pallas as pl
from jax.experimental.pallas import tpu as pltpu

M, K, N = 8192, 2048, 64
CHUNK = 512
NCHUNKS = M // CHUNK
NBUF = 8
NOBUF = 2


def _gate_kernel(x_hbm, w_hbm, o_hbm, xbuf, wbuf, obuf, insem, wsem, outsem):
    def in_copy(c, slot):
        return pltpu.make_async_copy(
            x_hbm.at[pl.ds(c * CHUNK, CHUNK), :],
            xbuf.at[slot],
            insem.at[slot],
        )

    def out_copy(c, oslot):
        return pltpu.make_async_copy(
            obuf.at[oslot],
            o_hbm.at[:, pl.ds(c * CHUNK, CHUNK)],
            outsem.at[oslot],
        )

    w_load = pltpu.make_async_copy(w_hbm, wbuf, wsem)
    w_load.start()
    for s in range(NBUF):
        in_copy(s, s).start()
    w_load.wait()

    w = wbuf[...].astype(jnp.bfloat16)
    for c in range(NCHUNKS):
        slot = c % NBUF
        oslot = c % NOBUF
        in_copy(c, slot).wait()
        if c >= NOBUF:
            out_copy(c - NOBUF, oslot).wait()
        obuf[oslot] = jax.lax.dot_general(
            w, xbuf[slot].astype(jnp.bfloat16), (((1,), (1,)), ((), ())),
            preferred_element_type=jnp.float32)
        out_copy(c, oslot).start()
        if c + NBUF < NCHUNKS:
            in_copy(c + NBUF, slot).start()

    for c in range(max(NCHUNKS - NOBUF, 0), NCHUNKS):
        out_copy(c, c % NOBUF).wait()


def kernel(x, W):
    out_t = pl.pallas_call(
        _gate_kernel,
        grid=(1,),
        in_specs=[
            pl.BlockSpec(memory_space=pltpu.MemorySpace.HBM),
            pl.BlockSpec(memory_space=pltpu.MemorySpace.HBM),
        ],
        out_specs=pl.BlockSpec(memory_space=pltpu.MemorySpace.HBM),
        out_shape=jax.ShapeDtypeStruct((N, M), jnp.float32),
        scratch_shapes=[
            pltpu.VMEM((NBUF, CHUNK, K), jnp.float32),
            pltpu.VMEM((N, K), jnp.float32),
            pltpu.VMEM((NOBUF, N, CHUNK), jnp.float32),
            pltpu.SemaphoreType.DMA((NBUF,)),
            pltpu.SemaphoreType.DMA,
            pltpu.SemaphoreType.DMA((NOBUF,)),
        ],
    )(x, W)
    return out_t.T
